# Initial kernel scaffold; baseline (speedup 1.0000x reference)
import jax
import jax.numpy as jnp
from jax.experimental import pallas as pl


def kernel(x_paper, x_author, edge_index_writes, edge_index_rev_writes, y_paper, y_author, W_paper, b_paper, W_author, b_author, W_lin, b_lin):
    raise NotImplementedError("write your pallas kernel here")


# speedup vs baseline: 2.7984x; 2.7984x over previous
import functools

import jax
import jax.numpy as jnp
from jax import lax
from jax.experimental import pallas as pl
from jax.experimental.pallas import tpu as pltpu
from jax.experimental.pallas import tpu_sc as plsc

N_PAPER = 50000
N_AUTHOR = 50000
E = 800000
D_IN = 128
D_HID = 64
D_OUT = 32

NC = 2
NS = 16
NW = NC * NS
GROUP = 128
GPB = 2
NSB = 100
EPW = GROUP * GPB * NSB
E_PAD = EPW * NW
N_ACC = 51200
RPT = N_ACC // NS
BLK = 400


def _proj_body(xp, xa, wp, wa, bp, ba, hp0, hp1, ha0, ha1):
    hp = jnp.dot(xp[...], wp[...], preferred_element_type=jnp.float32) + bp[...]
    ha = jnp.dot(xa[...], wa[...], preferred_element_type=jnp.float32) + ba[...]
    hp0[...] = hp[:, :32]
    hp1[...] = hp[:, 32:]
    ha0[...] = ha[:, :32]
    ha1[...] = ha[:, 32:]


def _project(x_paper, x_author, W_paper, b_paper, W_author, b_author):
    n = N_PAPER
    grid = n // BLK
    row = pl.BlockSpec((BLK, D_IN), lambda i: (i, 0))
    full = pl.BlockSpec((D_IN, D_HID), lambda i: (0, 0))
    bias = pl.BlockSpec((1, D_HID), lambda i: (0, 0))
    out = pl.BlockSpec((BLK, 32), lambda i: (i, 0))
    shp = jax.ShapeDtypeStruct((n, 32), jnp.float32)
    return pl.pallas_call(
        _proj_body,
        grid=(grid,),
        in_specs=[row, row, full, full, bias, bias],
        out_specs=[out, out, out, out],
        out_shape=[shp, shp, shp, shp],
    )(x_paper, x_author, W_paper, W_author,
      b_paper.reshape(1, D_HID), b_author.reshape(1, D_HID))


def _make_seg_sum(n_tables):
    mesh = plsc.VectorSubcoreMesh(core_axis_name="c", subcore_axis_name="s",
                                  num_cores=NC, num_subcores=NS)
    n_out = n_tables + 1
    out_type = [jax.ShapeDtypeStruct((NC, N_ACC, 32), jnp.float32)
                for _ in range(n_out)]
    scratch = [
        pltpu.VMEM_SHARED((N_ACC, 32), jnp.float32),
        pltpu.VMEM((2, GPB, GROUP), jnp.int32),
        pltpu.VMEM((2, GPB, GROUP), jnp.int32),
        pltpu.VMEM((2, GPB * GROUP, 32), jnp.float32),
        pltpu.VMEM((GROUP, 32), jnp.float32),
        pltpu.SemaphoreType.DMA,
    ]

    def body(*refs):
        src3d, dst3d, zeros_hbm, ones_hbm = refs[0], refs[1], refs[2], refs[3]
        tabs = refs[4:4 + n_tables]
        outs = refs[4 + n_tables:4 + n_tables + n_out]
        acc, sidx, didx, dbuf, obuf, gsem = refs[4 + n_tables + n_out:]

        cid = lax.axis_index("c")
        sid = lax.axis_index("s")
        wid = sid * NC + cid
        blkbase = wid * NSB

        pltpu.sync_copy(ones_hbm, obuf)

        def zero_own_rows():
            pltpu.sync_copy(zeros_hbm, acc.at[pl.ds(sid * RPT, RPT)])

        def dump(out):
            pltpu.sync_copy(acc.at[pl.ds(sid * RPT, RPT)],
                            out.at[cid, pl.ds(sid * RPT, RPT)])

        for t in range(n_tables):
            table = tabs[t]
            zero_own_rows()
            plsc.subcore_barrier()

            pltpu.sync_copy(src3d.at[blkbase], sidx.at[0])
            pltpu.sync_copy(dst3d.at[blkbase], didx.at[0])
            for j in range(GPB):
                pltpu.async_copy(table.at[sidx.at[0, j]],
                                 dbuf.at[0, pl.ds(j * GROUP, GROUP)], gsem)

            def step(sb, par):
                other = 1 - par
                for j in range(GPB):
                    pltpu.make_async_copy(
                        table.at[sidx.at[par, j]],
                        dbuf.at[par, pl.ds(j * GROUP, GROUP)], gsem).wait()

                @pl.when(sb + 1 < NSB)
                def _():
                    pltpu.sync_copy(src3d.at[blkbase + sb + 1], sidx.at[other])
                    pltpu.sync_copy(dst3d.at[blkbase + sb + 1], didx.at[other])
                    for j in range(GPB):
                        pltpu.async_copy(table.at[sidx.at[other, j]],
                                         dbuf.at[other, pl.ds(j * GROUP, GROUP)],
                                         gsem)

                for j in range(GPB):
                    pltpu.sync_copy(dbuf.at[par, pl.ds(j * GROUP, GROUP)],
                                    acc.at[didx.at[par, j]], add=True)

            def sb_pair(k, _):
                step(2 * k, 0)
                step(2 * k + 1, 1)
                return 0
            lax.fori_loop(0, NSB // 2, sb_pair, 0)

            plsc.subcore_barrier()
            dump(outs[t])

        zero_own_rows()
        plsc.subcore_barrier()

        def csb(sb, _):
            pltpu.sync_copy(dst3d.at[blkbase + sb], didx.at[0])
            for j in range(GPB):
                pltpu.sync_copy(obuf, acc.at[didx.at[0, j]], add=True)
            return 0
        lax.fori_loop(0, NSB, csb, 0)

        plsc.subcore_barrier()
        dump(outs[n_tables])

    return pl.kernel(
        body, out_type, mesh=mesh, scratch_types=scratch,
        compiler_params=pltpu.CompilerParams(use_tc_tiling_on_sc=False))


def _prep_edges(edge_index):
    src = edge_index[0].astype(jnp.int32)
    dst = edge_index[1].astype(jnp.int32)
    pad = E_PAD - E
    src = jnp.concatenate([src, jnp.zeros((pad,), jnp.int32)])
    dst = jnp.concatenate([dst, jnp.full((pad,), N_ACC - 1, jnp.int32)])
    shape = (E_PAD // (GPB * GROUP), GPB, GROUP)
    return src.reshape(shape), dst.reshape(shape)


def _norm_body(p0, p1, p2, pc, z0, z1, z2):
    cnt = pc[0, :, 0] + pc[1, :, 0]
    r = (1.0 / jnp.maximum(cnt, 1.0))[:, None]
    z0[...] = (p0[0] + p0[1]) * r
    z1[...] = (p1[0] + p1[1]) * r
    z2[...] = (p2[0] + p2[1]) * r


def _normalize3(P0, P1, P2, PC, n):
    grid = n // BLK
    part = pl.BlockSpec((NC, BLK, 32), lambda i: (0, i, 0))
    out = pl.BlockSpec((BLK, 32), lambda i: (i, 0))
    shp = jax.ShapeDtypeStruct((n, 32), jnp.float32)
    return pl.pallas_call(
        _norm_body,
        grid=(grid,),
        in_specs=[part, part, part, part],
        out_specs=[out, out, out],
        out_shape=[shp, shp, shp],
    )(P0, P1, P2, PC)


def _final_body(q0, q1, q2, q3, q4, q5, qc, wl, bl, res):
    cnt = qc[0, :, 0] + qc[1, :, 0]
    r = (1.0 / jnp.maximum(cnt, 1.0))[:, None]
    m1a = (q0[0] + q0[1]) * r
    m1b = (q1[0] + q1[1]) * r
    m2a = (q2[0] + q2[1]) * r
    m2b = (q3[0] + q3[1]) * r
    p1 = (q4[0] + q4[1]) * r
    p2 = (q5[0] + q5[1]) * r
    oa = 0.5 * (jnp.maximum(m1a, 0.0) + jnp.maximum(m2a, 0.0))
    ob = 0.5 * (jnp.maximum(m1b, 0.0) + jnp.maximum(m2b, 0.0))
    out = jnp.dot(oa, wl[:32, :], preferred_element_type=jnp.float32)
    out = out + jnp.dot(ob, wl[32:, :], preferred_element_type=jnp.float32)
    res[...] = 0.5 * (p1 + p2) + out + bl[...]


def _final(Q, QC, W_lin, b_lin):
    grid = N_PAPER // BLK
    part = pl.BlockSpec((NC, BLK, 32), lambda i: (0, i, 0))
    wspec = pl.BlockSpec((D_HID, D_OUT), lambda i: (0, 0))
    bspec = pl.BlockSpec((1, D_OUT), lambda i: (0, 0))
    out = pl.BlockSpec((BLK, D_OUT), lambda i: (i, 0))
    return pl.pallas_call(
        _final_body,
        grid=(grid,),
        in_specs=[part] * 7 + [wspec, bspec],
        out_specs=out,
        out_shape=jax.ShapeDtypeStruct((N_PAPER, D_OUT), jnp.float32),
    )(*Q, QC, W_lin, b_lin.reshape(1, D_OUT))


_make_seg_sum = functools.lru_cache(maxsize=None)(_make_seg_sum)


def kernel(x_paper, x_author, edge_index_writes, edge_index_rev_writes,
           y_paper, y_author, W_paper, b_paper, W_author, b_author,
           W_lin, b_lin):
    hp0, hp1, ha0, ha1 = _project(x_paper, x_author, W_paper, b_paper,
                                  W_author, b_author)
    src_r, dst_r = _prep_edges(edge_index_rev_writes)
    src_w, dst_w = _prep_edges(edge_index_writes)
    zeros_acc = jnp.zeros((RPT, 32), jnp.float32)
    ones_grp = jnp.ones((GROUP, 32), jnp.float32)

    P0, P1, P2, PC = _make_seg_sum(3)(src_r, dst_r, zeros_acc, ones_grp,
                                      hp0, hp1, y_paper)
    Z0, Z1, Zy = _normalize3(P0, P1, P2, PC, N_AUTHOR)

    Q = _make_seg_sum(6)(src_w, dst_w, zeros_acc, ones_grp,
                         ha0, ha1, Z0, Z1, y_author, Zy)
    return _final(Q[:6], Q[6], W_lin, b_lin)


# speedup vs baseline: 3.2864x; 1.1744x over previous
import functools

import jax
import jax.numpy as jnp
from jax import lax
from jax.experimental import pallas as pl
from jax.experimental.pallas import tpu as pltpu
from jax.experimental.pallas import tpu_sc as plsc

N_PAPER = 50000
N_AUTHOR = 50000
E = 800000
D_IN = 128
D_HID = 64
D_OUT = 32

NC = 2
NS = 16
NW = NC * NS
GROUP = 128
GPB = 2
NSB = 100
EPW = GROUP * GPB * NSB
E_PAD = EPW * NW
N_ACC = 51200
RPT = N_ACC // NS
BLK = 400


def _proj_body(xp, xa, wp, wa, bp, ba, hp0, hp1, ha0, ha1):
    hp = jnp.dot(xp[...], wp[...], preferred_element_type=jnp.float32) + bp[...]
    ha = jnp.dot(xa[...], wa[...], preferred_element_type=jnp.float32) + ba[...]
    hp0[...] = hp[:, :32]
    hp1[...] = hp[:, 32:]
    ha0[...] = ha[:, :32]
    ha1[...] = ha[:, 32:]


def _project(x_paper, x_author, W_paper, b_paper, W_author, b_author):
    n = N_PAPER
    grid = n // BLK
    row = pl.BlockSpec((BLK, D_IN), lambda i: (i, 0))
    full = pl.BlockSpec((D_IN, D_HID), lambda i: (0, 0))
    bias = pl.BlockSpec((1, D_HID), lambda i: (0, 0))
    out = pl.BlockSpec((BLK, 32), lambda i: (i, 0))
    shp = jax.ShapeDtypeStruct((n, 32), jnp.float32)
    return pl.pallas_call(
        _proj_body,
        grid=(grid,),
        in_specs=[row, row, full, full, bias, bias],
        out_specs=[out, out, out, out],
        out_shape=[shp, shp, shp, shp],
    )(x_paper, x_author, W_paper, W_author,
      b_paper.reshape(1, D_HID), b_author.reshape(1, D_HID))


def _make_seg_sum(n_tables):
    mesh = plsc.VectorSubcoreMesh(core_axis_name="c", subcore_axis_name="s",
                                  num_cores=NC, num_subcores=NS)
    n_out = n_tables + 1
    out_type = [jax.ShapeDtypeStruct((NC, N_ACC, 32), jnp.float32)
                for _ in range(n_out)]
    scratch = [
        pltpu.VMEM_SHARED((N_ACC, 32), jnp.float32),
        pltpu.VMEM((2, GPB, GROUP), jnp.int32),
        pltpu.VMEM((2, GPB, GROUP), jnp.int32),
        pltpu.VMEM((2, GPB * GROUP, 32), jnp.float32),
        pltpu.VMEM((GROUP, 32), jnp.float32),
        pltpu.SemaphoreType.DMA,
        pltpu.SemaphoreType.DMA,
        pltpu.SemaphoreType.DMA,
    ]

    def body(*refs):
        src3d, dst3d, zeros_hbm, ones_hbm = refs[0], refs[1], refs[2], refs[3]
        tabs = refs[4:4 + n_tables]
        outs = refs[4 + n_tables:4 + n_tables + n_out]
        (acc, sidx, didx, dbuf, obuf,
         gsem, ssem, isem) = refs[4 + n_tables + n_out:]

        cid = lax.axis_index("c")
        sid = lax.axis_index("s")
        wid = sid * NC + cid
        blkbase = wid * NSB

        pltpu.sync_copy(ones_hbm, obuf)

        def zero_own_rows():
            pltpu.sync_copy(zeros_hbm, acc.at[pl.ds(sid * RPT, RPT)])

        def dump(out):
            pltpu.sync_copy(acc.at[pl.ds(sid * RPT, RPT)],
                            out.at[cid, pl.ds(sid * RPT, RPT)])

        def gat(par, j, table):
            return pltpu.make_async_copy(
                table.at[sidx.at[par, j]],
                dbuf.at[par, pl.ds(j * GROUP, GROUP)], gsem)

        def sca(par, j):
            return pltpu.make_async_copy(
                dbuf.at[par, pl.ds(j * GROUP, GROUP)],
                acc.at[didx.at[par, j]], ssem)

        def idx_load(sb, par):
            pltpu.async_copy(src3d.at[blkbase + sb], sidx.at[par], isem)
            pltpu.async_copy(dst3d.at[blkbase + sb], didx.at[par], isem)

        def idx_wait(sb, par):
            pltpu.make_async_copy(src3d.at[blkbase + sb], sidx.at[par],
                                  isem).wait()
            pltpu.make_async_copy(dst3d.at[blkbase + sb], didx.at[par],
                                  isem).wait()

        for t in range(n_tables):
            table = tabs[t]
            zero_own_rows()
            plsc.subcore_barrier()

            idx_load(0, 0)
            idx_wait(0, 0)
            for j in range(GPB):
                gat(0, j, table).start()

            def step(sb, par):
                other = 1 - par

                @pl.when(sb >= 1)
                def _():
                    for j in range(GPB):
                        sca(other, j).wait()

                @pl.when(sb + 1 < NSB)
                def _():
                    idx_load(sb + 1, other)

                for j in range(GPB):
                    gat(par, j, table).wait()

                @pl.when(sb + 1 < NSB)
                def _():
                    idx_wait(sb + 1, other)
                    for j in range(GPB):
                        gat(other, j, table).start()

                for j in range(GPB):
                    sca(par, j).start(add=True)

            def sb_pair(k, _):
                step(2 * k, 0)
                step(2 * k + 1, 1)
                return 0
            lax.fori_loop(0, NSB // 2, sb_pair, 0)
            for j in range(GPB):
                sca(1, j).wait()

            plsc.subcore_barrier()
            dump(outs[t])

        zero_own_rows()
        plsc.subcore_barrier()

        def csca(par, j):
            return pltpu.make_async_copy(obuf, acc.at[didx.at[par, j]], ssem)

        idx_load(0, 0)
        idx_wait(0, 0)

        def cstep(sb, par):
            other = 1 - par

            @pl.when(sb >= 1)
            def _():
                for j in range(GPB):
                    csca(other, j).wait()

            @pl.when(sb + 1 < NSB)
            def _():
                idx_load(sb + 1, other)
                idx_wait(sb + 1, other)

            for j in range(GPB):
                csca(par, j).start(add=True)

        def csb_pair(k, _):
            cstep(2 * k, 0)
            cstep(2 * k + 1, 1)
            return 0
        lax.fori_loop(0, NSB // 2, csb_pair, 0)
        for j in range(GPB):
            csca(1, j).wait()

        plsc.subcore_barrier()
        dump(outs[n_tables])

    return pl.kernel(
        body, out_type, mesh=mesh, scratch_types=scratch,
        compiler_params=pltpu.CompilerParams(use_tc_tiling_on_sc=False))


def _prep_edges(edge_index):
    src = edge_index[0].astype(jnp.int32)
    dst = edge_index[1].astype(jnp.int32)
    pad = E_PAD - E
    src = jnp.concatenate([src, jnp.zeros((pad,), jnp.int32)])
    dst = jnp.concatenate([dst, jnp.full((pad,), N_ACC - 1, jnp.int32)])
    shape = (E_PAD // (GPB * GROUP), GPB, GROUP)
    return src.reshape(shape), dst.reshape(shape)


def _norm_body(p0, p1, p2, pc, z0, z1, z2):
    cnt = pc[0, :, 0] + pc[1, :, 0]
    r = (1.0 / jnp.maximum(cnt, 1.0))[:, None]
    z0[...] = (p0[0] + p0[1]) * r
    z1[...] = (p1[0] + p1[1]) * r
    z2[...] = (p2[0] + p2[1]) * r


def _normalize3(P0, P1, P2, PC, n):
    grid = n // BLK
    part = pl.BlockSpec((NC, BLK, 32), lambda i: (0, i, 0))
    out = pl.BlockSpec((BLK, 32), lambda i: (i, 0))
    shp = jax.ShapeDtypeStruct((n, 32), jnp.float32)
    return pl.pallas_call(
        _norm_body,
        grid=(grid,),
        in_specs=[part, part, part, part],
        out_specs=[out, out, out],
        out_shape=[shp, shp, shp],
    )(P0, P1, P2, PC)


def _final_body(q0, q1, q2, q3, q4, q5, qc, wl, bl, res):
    cnt = qc[0, :, 0] + qc[1, :, 0]
    r = (1.0 / jnp.maximum(cnt, 1.0))[:, None]
    m1a = (q0[0] + q0[1]) * r
    m1b = (q1[0] + q1[1]) * r
    m2a = (q2[0] + q2[1]) * r
    m2b = (q3[0] + q3[1]) * r
    p1 = (q4[0] + q4[1]) * r
    p2 = (q5[0] + q5[1]) * r
    oa = 0.5 * (jnp.maximum(m1a, 0.0) + jnp.maximum(m2a, 0.0))
    ob = 0.5 * (jnp.maximum(m1b, 0.0) + jnp.maximum(m2b, 0.0))
    out = jnp.dot(oa, wl[:32, :], preferred_element_type=jnp.float32)
    out = out + jnp.dot(ob, wl[32:, :], preferred_element_type=jnp.float32)
    res[...] = 0.5 * (p1 + p2) + out + bl[...]


def _final(Q, QC, W_lin, b_lin):
    grid = N_PAPER // BLK
    part = pl.BlockSpec((NC, BLK, 32), lambda i: (0, i, 0))
    wspec = pl.BlockSpec((D_HID, D_OUT), lambda i: (0, 0))
    bspec = pl.BlockSpec((1, D_OUT), lambda i: (0, 0))
    out = pl.BlockSpec((BLK, D_OUT), lambda i: (i, 0))
    return pl.pallas_call(
        _final_body,
        grid=(grid,),
        in_specs=[part] * 7 + [wspec, bspec],
        out_specs=out,
        out_shape=jax.ShapeDtypeStruct((N_PAPER, D_OUT), jnp.float32),
    )(*Q, QC, W_lin, b_lin.reshape(1, D_OUT))


_make_seg_sum = functools.lru_cache(maxsize=None)(_make_seg_sum)


def kernel(x_paper, x_author, edge_index_writes, edge_index_rev_writes,
           y_paper, y_author, W_paper, b_paper, W_author, b_author,
           W_lin, b_lin):
    hp0, hp1, ha0, ha1 = _project(x_paper, x_author, W_paper, b_paper,
                                  W_author, b_author)
    src_r, dst_r = _prep_edges(edge_index_rev_writes)
    src_w, dst_w = _prep_edges(edge_index_writes)
    zeros_acc = jnp.zeros((RPT, 32), jnp.float32)
    ones_grp = jnp.ones((GROUP, 32), jnp.float32)

    P0, P1, P2, PC = _make_seg_sum(3)(src_r, dst_r, zeros_acc, ones_grp,
                                      hp0, hp1, y_paper)
    Z0, Z1, Zy = _normalize3(P0, P1, P2, PC, N_AUTHOR)

    Q = _make_seg_sum(6)(src_w, dst_w, zeros_acc, ones_grp,
                         ha0, ha1, Z0, Z1, y_author, Zy)
    return _final(Q[:6], Q[6], W_lin, b_lin)


# speedup vs baseline: 3.6231x; 1.1025x over previous
import functools

import jax
import jax.numpy as jnp
from jax import lax
from jax.experimental import pallas as pl
from jax.experimental.pallas import tpu as pltpu
from jax.experimental.pallas import tpu_sc as plsc

N_PAPER = 50000
N_AUTHOR = 50000
E = 800000
D_IN = 128
D_HID = 64
D_OUT = 32

NC = 2
NS = 16
NW = NC * NS
GROUP = 128
GPB = 2
NSB = 100
NSB0 = 138
NSB1 = 62
E_PAD = GROUP * GPB * NS * (NSB0 + NSB1)
N_ACC = 51200
RPT = N_ACC // NS
BLK = 400


def _proj_body(xp, xa, wp, wa, bp, ba, hp0, hp1, ha0, ha1):
    hp = jnp.dot(xp[...], wp[...], preferred_element_type=jnp.float32) + bp[...]
    ha = jnp.dot(xa[...], wa[...], preferred_element_type=jnp.float32) + ba[...]
    hp0[...] = hp[:, :32]
    hp1[...] = hp[:, 32:]
    ha0[...] = ha[:, :32]
    ha1[...] = ha[:, 32:]


def _project(x_paper, x_author, W_paper, b_paper, W_author, b_author):
    n = N_PAPER
    grid = n // BLK
    row = pl.BlockSpec((BLK, D_IN), lambda i: (i, 0))
    full = pl.BlockSpec((D_IN, D_HID), lambda i: (0, 0))
    bias = pl.BlockSpec((1, D_HID), lambda i: (0, 0))
    out = pl.BlockSpec((BLK, 32), lambda i: (i, 0))
    shp = jax.ShapeDtypeStruct((n, 32), jnp.float32)
    return pl.pallas_call(
        _proj_body,
        grid=(grid,),
        in_specs=[row, row, full, full, bias, bias],
        out_specs=[out, out, out, out],
        out_shape=[shp, shp, shp, shp],
    )(x_paper, x_author, W_paper, W_author,
      b_paper.reshape(1, D_HID), b_author.reshape(1, D_HID))


def _make_seg_sum(n_tables, nsb0=NSB, nsb1=NSB):
    mesh = plsc.VectorSubcoreMesh(core_axis_name="c", subcore_axis_name="s",
                                  num_cores=NC, num_subcores=NS)
    n_out = n_tables + 1
    out_type = [jax.ShapeDtypeStruct((NC, N_ACC, 32), jnp.float32)
                for _ in range(n_out)]
    scratch = [
        pltpu.VMEM_SHARED((N_ACC, 32), jnp.float32),
        pltpu.VMEM((2, GPB, GROUP), jnp.int32),
        pltpu.VMEM((2, GPB, GROUP), jnp.int32),
        pltpu.VMEM((2, GPB * GROUP, 32), jnp.float32),
        pltpu.VMEM((GROUP, 32), jnp.float32),
        pltpu.SemaphoreType.DMA,
        pltpu.SemaphoreType.DMA,
        pltpu.SemaphoreType.DMA,
    ]

    def body(*refs):
        src3d, dst3d, zeros_hbm, ones_hbm = refs[0], refs[1], refs[2], refs[3]
        tabs = refs[4:4 + n_tables]
        outs = refs[4 + n_tables:4 + n_tables + n_out]
        (acc, sidx, didx, dbuf, obuf,
         gsem, ssem, isem) = refs[4 + n_tables + n_out:]

        cid = lax.axis_index("c")
        sid = lax.axis_index("s")
        nsb = jnp.where(cid == 0, nsb0, nsb1)
        blkbase = jnp.where(cid == 0, sid * nsb0, NS * nsb0 + sid * nsb1)

        pltpu.sync_copy(ones_hbm, obuf)

        def zero_own_rows():
            pltpu.sync_copy(zeros_hbm, acc.at[pl.ds(sid * RPT, RPT)])

        def dump(out):
            pltpu.sync_copy(acc.at[pl.ds(sid * RPT, RPT)],
                            out.at[cid, pl.ds(sid * RPT, RPT)])

        def gat(par, j, table):
            return pltpu.make_async_copy(
                table.at[sidx.at[par, j]],
                dbuf.at[par, pl.ds(j * GROUP, GROUP)], gsem)

        def sca(par, j):
            return pltpu.make_async_copy(
                dbuf.at[par, pl.ds(j * GROUP, GROUP)],
                acc.at[didx.at[par, j]], ssem)

        def idx_load(sb, par):
            pltpu.async_copy(src3d.at[blkbase + sb], sidx.at[par], isem)
            pltpu.async_copy(dst3d.at[blkbase + sb], didx.at[par], isem)

        def idx_wait(sb, par):
            pltpu.make_async_copy(src3d.at[blkbase + sb], sidx.at[par],
                                  isem).wait()
            pltpu.make_async_copy(dst3d.at[blkbase + sb], didx.at[par],
                                  isem).wait()

        for t in range(n_tables):
            table = tabs[t]
            zero_own_rows()
            plsc.subcore_barrier()

            idx_load(0, 0)
            idx_wait(0, 0)
            for j in range(GPB):
                gat(0, j, table).start()

            def step(sb, par):
                other = 1 - par

                @pl.when(sb >= 1)
                def _():
                    for j in range(GPB):
                        sca(other, j).wait()

                @pl.when(sb + 1 < nsb)
                def _():
                    idx_load(sb + 1, other)

                for j in range(GPB):
                    gat(par, j, table).wait()

                @pl.when(sb + 1 < nsb)
                def _():
                    idx_wait(sb + 1, other)
                    for j in range(GPB):
                        gat(other, j, table).start()

                for j in range(GPB):
                    sca(par, j).start(add=True)

            def sb_pair(k, _):
                step(2 * k, 0)
                step(2 * k + 1, 1)
                return 0
            lax.fori_loop(0, nsb // 2, sb_pair, 0)
            for j in range(GPB):
                sca(1, j).wait()

            plsc.subcore_barrier()
            dump(outs[t])

        zero_own_rows()
        plsc.subcore_barrier()

        def csca(par, j):
            return pltpu.make_async_copy(obuf, acc.at[didx.at[par, j]], ssem)

        idx_load(0, 0)
        idx_wait(0, 0)

        def cstep(sb, par):
            other = 1 - par

            @pl.when(sb >= 1)
            def _():
                for j in range(GPB):
                    csca(other, j).wait()

            @pl.when(sb + 1 < nsb)
            def _():
                idx_load(sb + 1, other)
                idx_wait(sb + 1, other)

            for j in range(GPB):
                csca(par, j).start(add=True)

        def csb_pair(k, _):
            cstep(2 * k, 0)
            cstep(2 * k + 1, 1)
            return 0
        lax.fori_loop(0, nsb // 2, csb_pair, 0)
        for j in range(GPB):
            csca(1, j).wait()

        plsc.subcore_barrier()
        dump(outs[n_tables])

    return pl.kernel(
        body, out_type, mesh=mesh, scratch_types=scratch,
        compiler_params=pltpu.CompilerParams(use_tc_tiling_on_sc=False))


def _prep_edges(edge_index):
    src = edge_index[0].astype(jnp.int32)
    dst = edge_index[1].astype(jnp.int32)
    pad = E_PAD - E
    src = jnp.concatenate([src, jnp.zeros((pad,), jnp.int32)])
    dst = jnp.concatenate([dst, jnp.full((pad,), N_ACC - 1, jnp.int32)])
    shape = (E_PAD // (GPB * GROUP), GPB, GROUP)
    return src.reshape(shape), dst.reshape(shape)


def _norm_body(p0, p1, p2, pc, z0, z1, z2):
    cnt = pc[0, :, 0] + pc[1, :, 0]
    r = (1.0 / jnp.maximum(cnt, 1.0))[:, None]
    z0[...] = (p0[0] + p0[1]) * r
    z1[...] = (p1[0] + p1[1]) * r
    z2[...] = (p2[0] + p2[1]) * r


def _normalize3(P0, P1, P2, PC, n):
    grid = n // BLK
    part = pl.BlockSpec((NC, BLK, 32), lambda i: (0, i, 0))
    out = pl.BlockSpec((BLK, 32), lambda i: (i, 0))
    shp = jax.ShapeDtypeStruct((n, 32), jnp.float32)
    return pl.pallas_call(
        _norm_body,
        grid=(grid,),
        in_specs=[part, part, part, part],
        out_specs=[out, out, out],
        out_shape=[shp, shp, shp],
    )(P0, P1, P2, PC)


def _final_body(q0, q1, q2, q3, q4, q5, qc, wl, bl, res):
    cnt = qc[0, :, 0] + qc[1, :, 0]
    r = (1.0 / jnp.maximum(cnt, 1.0))[:, None]
    m1a = (q0[0] + q0[1]) * r
    m1b = (q1[0] + q1[1]) * r
    m2a = (q2[0] + q2[1]) * r
    m2b = (q3[0] + q3[1]) * r
    p1 = (q4[0] + q4[1]) * r
    p2 = (q5[0] + q5[1]) * r
    oa = 0.5 * (jnp.maximum(m1a, 0.0) + jnp.maximum(m2a, 0.0))
    ob = 0.5 * (jnp.maximum(m1b, 0.0) + jnp.maximum(m2b, 0.0))
    out = jnp.dot(oa, wl[:32, :], preferred_element_type=jnp.float32)
    out = out + jnp.dot(ob, wl[32:, :], preferred_element_type=jnp.float32)
    res[...] = 0.5 * (p1 + p2) + out + bl[...]


def _final(Q, QC, W_lin, b_lin):
    grid = N_PAPER // BLK
    part = pl.BlockSpec((NC, BLK, 32), lambda i: (0, i, 0))
    wspec = pl.BlockSpec((D_HID, D_OUT), lambda i: (0, 0))
    bspec = pl.BlockSpec((1, D_OUT), lambda i: (0, 0))
    out = pl.BlockSpec((BLK, D_OUT), lambda i: (i, 0))
    return pl.pallas_call(
        _final_body,
        grid=(grid,),
        in_specs=[part] * 7 + [wspec, bspec],
        out_specs=out,
        out_shape=jax.ShapeDtypeStruct((N_PAPER, D_OUT), jnp.float32),
    )(*Q, QC, W_lin, b_lin.reshape(1, D_OUT))


_make_seg_sum = functools.lru_cache(maxsize=None)(_make_seg_sum)


def kernel(x_paper, x_author, edge_index_writes, edge_index_rev_writes,
           y_paper, y_author, W_paper, b_paper, W_author, b_author,
           W_lin, b_lin):
    hp0, hp1, ha0, ha1 = _project(x_paper, x_author, W_paper, b_paper,
                                  W_author, b_author)
    src_r, dst_r = _prep_edges(edge_index_rev_writes)
    src_w, dst_w = _prep_edges(edge_index_writes)
    zeros_acc = jnp.zeros((RPT, 32), jnp.float32)
    ones_grp = jnp.ones((GROUP, 32), jnp.float32)

    P0, P1, P2, PC = _make_seg_sum(3, NSB0, NSB1)(src_r, dst_r, zeros_acc,
                                                  ones_grp, hp0, hp1, y_paper)
    Z0, Z1, Zy = _normalize3(P0, P1, P2, PC, N_AUTHOR)

    Q = _make_seg_sum(6, NSB0, NSB1)(src_w, dst_w, zeros_acc, ones_grp,
                                     ha0, ha1, Z0, Z1, y_author, Zy)
    return _final(Q[:6], Q[6], W_lin, b_lin)
